# baseline (speedup 1.0000x reference)
import functools

import jax
import jax.numpy as jnp
from jax import lax
from jax.experimental import pallas as pl
from jax.experimental.pallas import tpu as pltpu
from jax.experimental.pallas import tpu_sc as plsc

B = 16384
D = 64

_info = plsc.get_sparse_core_info()
_NC = _info.num_cores
_NS = _info.num_subcores
_NW = _NC * _NS
_BPW = B // _NW


def _sc_gather4(uidx_hbm, iidx_hbm, t_umlp, t_imlp, t_umf, t_imf,
                o_umlp, o_imlp, o_umf, o_imf,
                uidx_v, iidx_v, rows_a, rows_b, sem_a, sem_b):
    wid = lax.axis_index("s") * _NC + lax.axis_index("c")
    base = wid * _BPW
    pltpu.sync_copy(uidx_hbm.at[pl.ds(base, _BPW)], uidx_v)
    pltpu.sync_copy(iidx_hbm.at[pl.ds(base, _BPW)], iidx_v)

    cp_a = pltpu.async_copy(t_umlp.at[uidx_v], rows_a, sem_a)
    cp_b = pltpu.async_copy(t_imlp.at[iidx_v], rows_b, sem_b)
    cp_a.wait()
    pltpu.sync_copy(rows_a, o_umlp.at[pl.ds(base, _BPW)])
    cp_a = pltpu.async_copy(t_umf.at[uidx_v], rows_a, sem_a)
    cp_b.wait()
    pltpu.sync_copy(rows_b, o_imlp.at[pl.ds(base, _BPW)])
    cp_b = pltpu.async_copy(t_imf.at[iidx_v], rows_b, sem_b)
    cp_a.wait()
    pltpu.sync_copy(rows_a, o_umf.at[pl.ds(base, _BPW)])
    cp_b.wait()
    pltpu.sync_copy(rows_b, o_imf.at[pl.ds(base, _BPW)])


_gather4 = functools.partial(
    pl.kernel,
    mesh=plsc.VectorSubcoreMesh(core_axis_name="c", subcore_axis_name="s"),
    out_type=[jax.ShapeDtypeStruct((B, D), jnp.float32)] * 4,
    scratch_types=[
        pltpu.VMEM((_BPW,), jnp.int32),
        pltpu.VMEM((_BPW,), jnp.int32),
        pltpu.VMEM((_BPW, D), jnp.float32),
        pltpu.VMEM((_BPW, D), jnp.float32),
        pltpu.SemaphoreType.DMA,
        pltpu.SemaphoreType.DMA,
    ],
    compiler_params=pltpu.CompilerParams(use_tc_tiling_on_sc=False),
)(_sc_gather4)


_BS = 2048


def _mlp_body(umlp, imlp, umf, imf, w0a, w0b, b0, w1, b1, w2, b2,
              wtop, wbot, bout, out):
    h = jnp.dot(umlp[...], w0a[...], preferred_element_type=jnp.float32)
    h += jnp.dot(imlp[...], w0b[...], preferred_element_type=jnp.float32)
    h = jnp.maximum(h + b0[...], 0.0)
    h = jnp.maximum(
        jnp.dot(h, w1[...], preferred_element_type=jnp.float32) + b1[...], 0.0)
    h = jnp.maximum(
        jnp.dot(h, w2[...], preferred_element_type=jnp.float32) + b2[...], 0.0)
    mf = umf[...] * imf[...]
    logits = jnp.dot(h, wtop[...], preferred_element_type=jnp.float32)
    logits += jnp.dot(mf, wbot[...], preferred_element_type=jnp.float32)
    logits += bout[...]
    out[...] = jax.nn.sigmoid(logits)


def _mlp_tower(umlp, imlp, umf, imf, W0, b0, W1, b1, W2, b2, W_out, b_out):
    w0a = W0[:D]
    w0b = W0[D:]
    wtop = W_out[:16]
    wbot = W_out[16:]
    grid = B // _BS
    row_spec = pl.BlockSpec((_BS, D), lambda i: (i, 0))
    full = lambda a: pl.BlockSpec(a.shape, lambda i: (0,) * a.ndim)
    args = (umlp, imlp, umf, imf, w0a, w0b, b0.reshape(1, -1), W1,
            b1.reshape(1, -1), W2, b2.reshape(1, -1), wtop, wbot,
            b_out.reshape(1, 1))
    specs = [row_spec, row_spec, row_spec, row_spec] + [full(a) for a in args[4:]]
    return pl.pallas_call(
        _mlp_body,
        grid=(grid,),
        in_specs=specs,
        out_specs=pl.BlockSpec((_BS, 1), lambda i: (i, 0)),
        out_shape=jax.ShapeDtypeStruct((B, 1), jnp.float32),
    )(*args)


def kernel(user_indices, item_indices, emb_user_mlp, emb_item_mlp,
           emb_user_mf, emb_item_mf, W0, b0, W1, b1, W2, b2, W_out, b_out):
    umlp, imlp, umf, imf = _gather4(
        user_indices.astype(jnp.int32), item_indices.astype(jnp.int32),
        emb_user_mlp, emb_item_mlp, emb_user_mf, emb_item_mf)
    return _mlp_tower(umlp, imlp, umf, imf, W0, b0, W1, b1, W2, b2,
                      W_out, b_out)
